# Initial kernel scaffold; baseline (speedup 1.0000x reference)
import jax
import jax.numpy as jnp
from jax.experimental import pallas as pl


def kernel(x, table):
    raise NotImplementedError("write your pallas kernel here")


# speedup vs baseline: 1.8254x; 1.8254x over previous
import functools

import jax
import jax.numpy as jnp
from jax import lax
from jax.experimental import pallas as pl
from jax.experimental.pallas import tpu as pltpu
from jax.experimental.pallas import tpu_sc as plsc

_V = 8192
_D = 8192
_B = 8192
_NC = 2
_NS = 16
_NW = _NC * _NS
_BW = _B // _NW
_R = 8
_NCHUNK = _BW // _R


def _gather_body(table_hbm, idx_hbm, out_hbm, idx_v, buf, sem):
    wid = lax.axis_index("s") * _NC + lax.axis_index("c")
    base = wid * _BW
    pltpu.sync_copy(idx_hbm.at[pl.ds(base, _BW)], idx_v)

    @pl.loop(0, _NCHUNK)
    def _chunk(i):
        off = pl.multiple_of(i * _R, _R)
        idx_c = idx_v.at[pl.ds(off, _R)]
        pltpu.async_copy(table_hbm.at[idx_c], buf, sem).wait()
        pltpu.sync_copy(buf, out_hbm.at[pl.ds(base + off, _R)])


@jax.jit
def _gather(table, idx):
    run = functools.partial(
        pl.kernel,
        mesh=plsc.VectorSubcoreMesh(core_axis_name="c", subcore_axis_name="s"),
        out_type=jax.ShapeDtypeStruct((_B, _D), jnp.float32),
        scratch_types=[
            pltpu.VMEM((_BW,), jnp.int32),
            pltpu.VMEM((_R, _D), jnp.float32),
            pltpu.SemaphoreType.DMA,
        ],
    )(_gather_body)
    return run(table, idx)


def kernel(x, table):
    idx = x.reshape(-1)
    out = _gather(table, idx)
    return out.reshape(x.shape + (table.shape[1],))


# speedup vs baseline: 1.9358x; 1.0605x over previous
import functools

import jax
import jax.numpy as jnp
from jax import lax
from jax.experimental import pallas as pl
from jax.experimental.pallas import tpu as pltpu
from jax.experimental.pallas import tpu_sc as plsc

_V = 8192
_D = 8192
_B = 8192
_NC = 2
_NS = 16
_NW = _NC * _NS
_BW = _B // _NW
_R = 4
_NCHUNK = _BW // _R


def _gather_body(table_hbm, idx_hbm, out_hbm, idx_v, buf_a, buf_b,
                 ga, gb, oa, ob):
    wid = lax.axis_index("s") * _NC + lax.axis_index("c")
    base = wid * _BW
    pltpu.sync_copy(idx_hbm.at[wid], idx_v)

    def gather(chunk, buf, sem):
        pltpu.async_copy(table_hbm.at[idx_v.at[chunk]], buf, sem)

    def put(chunk, buf, sem):
        off = pl.multiple_of(chunk * _R, _R)
        pltpu.async_copy(buf, out_hbm.at[pl.ds(base + off, _R)], sem)

    def wait_gather(buf, sem):
        pltpu.make_async_copy(table_hbm.at[pl.ds(0, _R)], buf, sem).wait()

    def wait_put(buf, sem):
        pltpu.make_async_copy(buf, out_hbm.at[pl.ds(base, _R)], sem).wait()

    gather(0, buf_a, ga)
    wait_gather(buf_a, ga)
    put(0, buf_a, oa)
    gather(1, buf_b, gb)

    @pl.loop(1, _NCHUNK - 2, step=2)
    def _body(i):
        wait_gather(buf_b, gb)
        put(i, buf_b, ob)
        wait_put(buf_a, oa)
        gather(i + 1, buf_a, ga)
        wait_gather(buf_a, ga)
        put(i + 1, buf_a, oa)
        wait_put(buf_b, ob)
        gather(i + 2, buf_b, gb)

    wait_gather(buf_b, gb)
    put(_NCHUNK - 1, buf_b, ob)
    wait_put(buf_a, oa)
    wait_put(buf_b, ob)


@jax.jit
def _gather(table, idx):
    run = functools.partial(
        pl.kernel,
        mesh=plsc.VectorSubcoreMesh(core_axis_name="c", subcore_axis_name="s"),
        out_type=jax.ShapeDtypeStruct((_B, _D), jnp.float32),
        scratch_types=[
            pltpu.VMEM((_NCHUNK, _R), jnp.int32),
            pltpu.VMEM((_R, _D), jnp.float32),
            pltpu.VMEM((_R, _D), jnp.float32),
            pltpu.SemaphoreType.DMA,
            pltpu.SemaphoreType.DMA,
            pltpu.SemaphoreType.DMA,
            pltpu.SemaphoreType.DMA,
        ],
    )(_gather_body)
    return run(table, idx)


def kernel(x, table):
    idx = x.reshape(_NW, _NCHUNK, _R)
    out = _gather(table, idx)
    return out.reshape(x.shape + (table.shape[1],))
